# baseline (speedup 1.0000x reference)
import dataclasses
import functools

import jax
import jax.numpy as jnp
from jax import lax
from jax.experimental import pallas as pl
from jax.experimental.pallas import tpu as pltpu
from jax.experimental.pallas import tpu_sc as plsc

H = 128
N = 10000
E = 320000
NPAD = 10240
GAMMA = -0.45
ZETA = 1.05
NC = 2
NS = 16
NW = NC * NS
EPW = E // NW
CH = 80
NCH = EPW // CH
RB = 1000

_mesh = plsc.VectorSubcoreMesh(
    core_axis_name="c", subcore_axis_name="s", num_cores=NC, num_subcores=NS)

_sc_params = pltpu.CompilerParams(
    needs_layout_passes=False, use_tc_tiling_on_sc=False)


def _uv_body(x_ref, wnb_ref, bnb_ref, wself_ref, bself_ref, wa1_ref,
             wa2_ref, batt_ref, u_ref, v_ref):
    x = x_ref[...]
    a = jnp.maximum(x @ wnb_ref[...] + bnb_ref[...], 0.0)
    b = jnp.maximum(x @ wself_ref[...] + bself_ref[...], 0.0)
    u_ref[...] = a @ wa1_ref[...]
    v_ref[...] = b @ wa2_ref[...] + batt_ref[...]


def _tc_uv(x, Wnb, bnb, Wself, bself, Watt, batt):
    wa1 = Watt[:H, :]
    wa2 = Watt[H:, :]
    u, v = pl.pallas_call(
        _uv_body,
        grid=(N // RB,),
        in_specs=[
            pl.BlockSpec((RB, H), lambda i: (i, 0)),
            pl.BlockSpec((H, H), lambda i: (0, 0)),
            pl.BlockSpec((1, H), lambda i: (0, 0)),
            pl.BlockSpec((H, H), lambda i: (0, 0)),
            pl.BlockSpec((1, H), lambda i: (0, 0)),
            pl.BlockSpec((H, 1), lambda i: (0, 0)),
            pl.BlockSpec((H, 1), lambda i: (0, 0)),
            pl.BlockSpec((1, 1), lambda i: (0, 0)),
        ],
        out_specs=[
            pl.BlockSpec((RB, 1), lambda i: (i, 0)),
            pl.BlockSpec((RB, 1), lambda i: (i, 0)),
        ],
        out_shape=[
            jax.ShapeDtypeStruct((N, 1), jnp.float32),
            jax.ShapeDtypeStruct((N, 1), jnp.float32),
        ],
    )(x, Wnb, bnb.reshape(1, H), Wself, bself.reshape(1, H), wa1, wa2,
      batt.reshape(1, 1))
    return u.reshape(N), v.reshape(N)


def _comb_body(ya_ref, yb_ref, prev_ref, x_ref, out_ref):
    xn = jnp.concatenate(
        [ya_ref[0] + ya_ref[1], yb_ref[0] + yb_ref[1]], axis=-1)
    x_ref[...] = xn
    out_ref[...] = prev_ref[...] + xn


def _tc_combine(ya, yb, prev_out):
    return pl.pallas_call(
        _comb_body,
        grid=(N // RB,),
        in_specs=[
            pl.BlockSpec((NC, RB, HH), lambda i: (0, i, 0)),
            pl.BlockSpec((NC, RB, HH), lambda i: (0, i, 0)),
            pl.BlockSpec((RB, H), lambda i: (i, 0)),
        ],
        out_specs=[
            pl.BlockSpec((RB, H), lambda i: (i, 0)),
            pl.BlockSpec((RB, H), lambda i: (i, 0)),
        ],
        out_shape=[
            jax.ShapeDtypeStruct((N, H), jnp.float32),
            jax.ShapeDtypeStruct((N, H), jnp.float32),
        ],
    )(ya, yb, prev_out)


def _comb_uv_body(ya_ref, yb_ref, prev_ref, wnb_ref, bnb_ref, wself_ref,
                  bself_ref, wa1_ref, wa2_ref, batt_ref,
                  xa_ref, xb_ref, out_ref, u_ref, v_ref):
    xa = ya_ref[0] + ya_ref[1]
    xb = yb_ref[0] + yb_ref[1]
    xa_ref[...] = xa
    xb_ref[...] = xb
    xn = jnp.concatenate([xa, xb], axis=-1)
    out_ref[...] = prev_ref[...] + xn
    a = jnp.maximum(xn @ wnb_ref[...] + bnb_ref[...], 0.0)
    b = jnp.maximum(xn @ wself_ref[...] + bself_ref[...], 0.0)
    u_ref[...] = a @ wa1_ref[...]
    v_ref[...] = b @ wa2_ref[...] + batt_ref[...]


def _tc_comb_uv(ya, yb, prev_out, Wnb, bnb, Wself, bself, Watt, batt):
    wa1 = Watt[:H, :]
    wa2 = Watt[H:, :]
    full = lambda i: (0, 0)
    xa, xb, out, u, v = pl.pallas_call(
        _comb_uv_body,
        grid=(N // RB,),
        in_specs=[
            pl.BlockSpec((NC, RB, HH), lambda i: (0, i, 0)),
            pl.BlockSpec((NC, RB, HH), lambda i: (0, i, 0)),
            pl.BlockSpec((RB, H), lambda i: (i, 0)),
            pl.BlockSpec((H, H), full),
            pl.BlockSpec((1, H), full),
            pl.BlockSpec((H, H), full),
            pl.BlockSpec((1, H), full),
            pl.BlockSpec((H, 1), full),
            pl.BlockSpec((H, 1), full),
            pl.BlockSpec((1, 1), full),
        ],
        out_specs=[
            pl.BlockSpec((RB, HH), lambda i: (i, 0)),
            pl.BlockSpec((RB, HH), lambda i: (i, 0)),
            pl.BlockSpec((RB, H), lambda i: (i, 0)),
            pl.BlockSpec((RB, 1), lambda i: (i, 0)),
            pl.BlockSpec((RB, 1), lambda i: (i, 0)),
        ],
        out_shape=[
            jax.ShapeDtypeStruct((N, HH), jnp.float32),
            jax.ShapeDtypeStruct((N, HH), jnp.float32),
            jax.ShapeDtypeStruct((N, H), jnp.float32),
            jax.ShapeDtypeStruct((N, 1), jnp.float32),
            jax.ShapeDtypeStruct((N, 1), jnp.float32),
        ],
    )(ya, yb, prev_out, Wnb, bnb.reshape(1, H), Wself, bself.reshape(1, H),
      wa1, wa2, batt.reshape(1, 1))
    return xa, xb, out, u.reshape(N), v.reshape(N)


@functools.partial(
    pl.kernel,
    out_type=[
        jax.ShapeDtypeStruct((NW, NCH, CH), jnp.float32),
        jax.ShapeDtypeStruct((NC, NPAD), jnp.float32),
    ],
    mesh=_mesh,
    scratch_types=[
        pltpu.VMEM((N,), jnp.float32),
        pltpu.VMEM((N,), jnp.float32),
        pltpu.VMEM((NCH, CH), jnp.int32),
        pltpu.VMEM((NCH, CH), jnp.int32),
        pltpu.VMEM((NCH, CH), jnp.float32),
        pltpu.VMEM((NPAD // NS,), jnp.float32),
        pltpu.VMEM_SHARED((NPAD,), jnp.float32),
        pltpu.SemaphoreType.DMA,
    ],
    compiler_params=_sc_params,
)
def _sc_mask(u_hbm, v_hbm, row_hbm, col_hbm, mask_hbm, rs_hbm,
             u_v, v_v, row_v, col_v, mask_v, z_v, rs_sh, rsem):
    c = lax.axis_index("c")
    s = lax.axis_index("s")
    w = c * NS + s
    zn = NPAD // NS

    pltpu.sync_copy(u_hbm, u_v)
    pltpu.sync_copy(v_hbm, v_v)
    pltpu.sync_copy(row_hbm.at[w], row_v)
    pltpu.sync_copy(col_hbm.at[w], col_v)

    @pl.loop(0, zn, step=16)
    def _(i):
        z_v[pl.ds(i, 16)] = jnp.zeros((16,), jnp.float32)

    pltpu.sync_copy(z_v, rs_sh.at[pl.ds(s * zn, zn)])
    plsc.subcore_barrier()

    @pl.loop(0, NCH)
    def _(k):
        @plsc.parallel_loop(0, CH, step=16, unroll=2)
        def _(j):
            r = row_v[k, pl.ds(j, 16)]
            cl = col_v[k, pl.ds(j, 16)]
            ug = plsc.load_gather(u_v, [r])
            vg = plsc.load_gather(v_v, [cl])
            gate = 1.0 / (1.0 + jnp.exp(-(ug + vg)))
            m = gate * (ZETA - GAMMA) + GAMMA
            m = jnp.minimum(jnp.maximum(m, 0.0), 1.0)
            mask_v[k, pl.ds(j, 16)] = m
        pltpu.async_copy(mask_v.at[k], rs_sh.at[row_v.at[k]], rsem, add=True)

    @pl.loop(0, NCH)
    def _(k):
        pltpu.make_async_copy(
            mask_v.at[0], rs_sh.at[row_v.at[0]], rsem).wait()

    plsc.subcore_barrier()
    pltpu.sync_copy(rs_sh.at[pl.ds(s * zn, zn)], rs_hbm.at[c, pl.ds(s * zn, zn)])
    pltpu.sync_copy(mask_v, mask_hbm.at[w])


HH = H // 2


@functools.partial(
    pl.kernel,
    out_type=[
        jax.ShapeDtypeStruct((NC, NPAD, HH), jnp.float32),
        jax.ShapeDtypeStruct((NC, NPAD, HH), jnp.float32),
    ],
    mesh=_mesh,
    scratch_types=[
        pltpu.VMEM((NCH, CH), jnp.int32),
        pltpu.VMEM((NCH, CH), jnp.int32),
        pltpu.VMEM((NCH, CH), jnp.float32),
        pltpu.VMEM((NPAD,), jnp.float32),
        pltpu.VMEM((NPAD // NS,), jnp.float32),
        pltpu.VMEM((NPAD // NS,), jnp.float32),
        pltpu.VMEM_SHARED((NPAD,), jnp.float32),
        pltpu.VMEM((CH, HH), jnp.float32),
        pltpu.VMEM((CH, HH), jnp.float32),
        pltpu.VMEM((128, HH), jnp.float32),
        pltpu.VMEM_SHARED((NPAD, HH), jnp.float32),
        pltpu.SemaphoreType.DMA,
        pltpu.SemaphoreType.DMA,
        pltpu.SemaphoreType.DMA,
        pltpu.SemaphoreType.DMA,
    ],
    compiler_params=_sc_params,
)
def _sc_spmm(xa_hbm, xb_hbm, rs_hbm, coef_hbm, row_hbm, col_hbm,
             ya_hbm, yb_hbm, row_v, col_v, coef_v, d_v, rs0_v, rs1_v, d_sh,
             rows0_v, rows1_v, z_v, acc, gs0, gs1, ss0, ss1):
    c = lax.axis_index("c")
    s = lax.axis_index("s")
    w = c * NS + s
    rpw = NPAD // NS

    pltpu.sync_copy(row_hbm.at[w], row_v)
    pltpu.sync_copy(col_hbm.at[w], col_v)
    pltpu.sync_copy(coef_hbm.at[w], coef_v)

    pltpu.sync_copy(rs_hbm.at[0, pl.ds(s * rpw, rpw)], rs0_v)
    pltpu.sync_copy(rs_hbm.at[1, pl.ds(s * rpw, rpw)], rs1_v)

    @plsc.parallel_loop(0, rpw, step=16, unroll=2)
    def _(i):
        sl = pl.ds(i, 16)
        rs = rs0_v[sl] + rs1_v[sl] + 1e-6
        yi = plsc.bitcast(
            jnp.int32(0x5F3759DF) - (plsc.bitcast(rs, jnp.int32) >> 1),
            jnp.float32)
        for _ in range(3):
            yi = yi * (1.5 - 0.5 * rs * yi * yi)
        rs0_v[sl] = jnp.minimum(yi, 10.0)

    pltpu.sync_copy(rs0_v, d_sh.at[pl.ds(s * rpw, rpw)])
    plsc.subcore_barrier()
    pltpu.sync_copy(d_sh, d_v)

    @pl.loop(0, 128)
    def _(i):
        @pl.loop(0, HH, step=16)
        def _(j):
            z_v[i, pl.ds(j, 16)] = jnp.zeros((16,), jnp.float32)

    @pl.loop(0, NCH)
    def _(k):
        @plsc.parallel_loop(0, CH, step=16, unroll=2)
        def _(j):
            r = row_v[k, pl.ds(j, 16)]
            cl = col_v[k, pl.ds(j, 16)]
            dg = plsc.load_gather(d_v, [r]) * plsc.load_gather(d_v, [cl])
            coef_v[k, pl.ds(j, 16)] = coef_v[k, pl.ds(j, 16)] * dg

    def _scale(rows_ref, kk):
        fk = jnp.full((16,), kk, jnp.int32)

        @plsc.parallel_loop(0, CH, step=8, unroll=4)
        def _(e):
            for de in range(8):
                sv = plsc.load_gather(
                    coef_v, [fk, jnp.full((16,), e + de, jnp.int32)])
                for jj in range(HH // 16):
                    sl = pl.ds(jj * 16, 16)
                    rows_ref[e + de, sl] = rows_ref[e + de, sl] * sv

    bufs = ((rows0_v, gs0, ss0), (rows1_v, gs1, ss1))

    def _half(x_hbm, y_hbm):
        for t in range(5):
            pltpu.sync_copy(z_v, acc.at[pl.ds(s * rpw + t * 128, 128), :])
        plsc.subcore_barrier()

        for b, (rows_b, gs_b, ss_b) in enumerate(bufs):
            pltpu.async_copy(x_hbm.at[col_v.at[b]], rows_b, gs_b)

        @pl.loop(0, NCH - 1, step=2)
        def _(k):
            for b, (rows_b, gs_b, ss_b) in enumerate(bufs):
                kk = k + b
                pltpu.make_async_copy(
                    x_hbm.at[col_v.at[kk]], rows_b, gs_b).wait()
                _scale(rows_b, kk)
                pltpu.async_copy(rows_b, acc.at[row_v.at[kk]], ss_b, add=True)

                @pl.when(kk + 2 < NCH)
                def _():
                    pltpu.make_async_copy(
                        rows_b, acc.at[row_v.at[kk]], ss_b).wait()
                    pltpu.async_copy(x_hbm.at[col_v.at[kk + 2]], rows_b, gs_b)

        kk = NCH - 1
        pltpu.make_async_copy(x_hbm.at[col_v.at[kk]], rows0_v, gs0).wait()
        _scale(rows0_v, kk)
        pltpu.async_copy(rows0_v, acc.at[row_v.at[kk]], ss0, add=True)
        pltpu.make_async_copy(rows0_v, acc.at[row_v.at[0]], ss0).wait()
        pltpu.make_async_copy(rows1_v, acc.at[row_v.at[0]], ss1).wait()

        plsc.subcore_barrier()
        for t in range(5):
            sl = pl.ds(s * rpw + t * 128, 128)
            pltpu.sync_copy(acc.at[sl, :], y_hbm.at[c, sl, :])

    _half(xa_hbm, ya_hbm)
    plsc.subcore_barrier()
    _half(xb_hbm, yb_hbm)


def kernel(features, edge_index, Wnb0, bnb0, Wself0, bself0, Watt0, batt0,
           Wnb1, bnb1, Wself1, bself1, Watt1, batt1):
    row3 = edge_index[0].reshape(NW, NCH, CH)
    col3 = edge_index[1].reshape(NW, NCH, CH)
    u, v = _tc_uv(features, Wnb0, bnb0, Wself0, bself0, Watt0, batt0)
    mask3, rs = _sc_mask(u, v, row3, col3)
    ya, yb = _sc_spmm(features[:, :HH], features[:, HH:], rs, mask3,
                      row3, col3)
    xa, xb, out, u2, v2 = _tc_comb_uv(ya, yb, features, Wnb1, bnb1,
                                      Wself1, bself1, Watt1, batt1)
    mask3, rs = _sc_mask(u2, v2, row3, col3)
    ya, yb = _sc_spmm(xa, xb, rs, mask3, row3, col3)
    _, out = _tc_combine(ya, yb, out)
    return out


# speedup vs baseline: 1.1377x; 1.1377x over previous
import dataclasses
import functools

import jax
import jax.numpy as jnp
from jax import lax
from jax.experimental import pallas as pl
from jax.experimental.pallas import tpu as pltpu
from jax.experimental.pallas import tpu_sc as plsc

H = 128
N = 10000
E = 320000
NPAD = 10240
GAMMA = -0.45
ZETA = 1.05
NC = 2
NS = 16
NW = NC * NS
EPW = E // NW
CH = 80
NCH = EPW // CH
RB = 1000

_mesh = plsc.VectorSubcoreMesh(
    core_axis_name="c", subcore_axis_name="s", num_cores=NC, num_subcores=NS)

_sc_params = pltpu.CompilerParams(
    needs_layout_passes=False, use_tc_tiling_on_sc=False)


def _uv_body(x_ref, wnb_ref, bnb_ref, wself_ref, bself_ref, wa1_ref,
             wa2_ref, batt_ref, u_ref, v_ref):
    x = x_ref[...]
    a = jnp.maximum(x @ wnb_ref[...] + bnb_ref[...], 0.0)
    b = jnp.maximum(x @ wself_ref[...] + bself_ref[...], 0.0)
    u_ref[...] = a @ wa1_ref[...]
    v_ref[...] = b @ wa2_ref[...] + batt_ref[...]


def _tc_uv(x, Wnb, bnb, Wself, bself, Watt, batt):
    wa1 = Watt[:H, :]
    wa2 = Watt[H:, :]
    u, v = pl.pallas_call(
        _uv_body,
        grid=(N // RB,),
        in_specs=[
            pl.BlockSpec((RB, H), lambda i: (i, 0)),
            pl.BlockSpec((H, H), lambda i: (0, 0)),
            pl.BlockSpec((1, H), lambda i: (0, 0)),
            pl.BlockSpec((H, H), lambda i: (0, 0)),
            pl.BlockSpec((1, H), lambda i: (0, 0)),
            pl.BlockSpec((H, 1), lambda i: (0, 0)),
            pl.BlockSpec((H, 1), lambda i: (0, 0)),
            pl.BlockSpec((1, 1), lambda i: (0, 0)),
        ],
        out_specs=[
            pl.BlockSpec((RB, 1), lambda i: (i, 0)),
            pl.BlockSpec((RB, 1), lambda i: (i, 0)),
        ],
        out_shape=[
            jax.ShapeDtypeStruct((N, 1), jnp.float32),
            jax.ShapeDtypeStruct((N, 1), jnp.float32),
        ],
    )(x, Wnb, bnb.reshape(1, H), Wself, bself.reshape(1, H), wa1, wa2,
      batt.reshape(1, 1))
    return u.reshape(N), v.reshape(N)


def _comb_body(ya_ref, yb_ref, prev_ref, x_ref, out_ref):
    xn = jnp.concatenate(
        [ya_ref[0] + ya_ref[1], yb_ref[0] + yb_ref[1]], axis=-1)
    x_ref[...] = xn
    out_ref[...] = prev_ref[...] + xn


def _tc_combine(ya, yb, prev_out):
    return pl.pallas_call(
        _comb_body,
        grid=(N // RB,),
        in_specs=[
            pl.BlockSpec((NC, RB, HH), lambda i: (0, i, 0)),
            pl.BlockSpec((NC, RB, HH), lambda i: (0, i, 0)),
            pl.BlockSpec((RB, H), lambda i: (i, 0)),
        ],
        out_specs=[
            pl.BlockSpec((RB, H), lambda i: (i, 0)),
            pl.BlockSpec((RB, H), lambda i: (i, 0)),
        ],
        out_shape=[
            jax.ShapeDtypeStruct((N, H), jnp.float32),
            jax.ShapeDtypeStruct((N, H), jnp.float32),
        ],
    )(ya, yb, prev_out)


def _comb_uv_body(ya_ref, yb_ref, prev_ref, wnb_ref, bnb_ref, wself_ref,
                  bself_ref, wa1_ref, wa2_ref, batt_ref,
                  xa_ref, xb_ref, out_ref, u_ref, v_ref):
    xa = ya_ref[0] + ya_ref[1]
    xb = yb_ref[0] + yb_ref[1]
    xa_ref[...] = xa
    xb_ref[...] = xb
    xn = jnp.concatenate([xa, xb], axis=-1)
    out_ref[...] = prev_ref[...] + xn
    a = jnp.maximum(xn @ wnb_ref[...] + bnb_ref[...], 0.0)
    b = jnp.maximum(xn @ wself_ref[...] + bself_ref[...], 0.0)
    u_ref[...] = a @ wa1_ref[...]
    v_ref[...] = b @ wa2_ref[...] + batt_ref[...]


def _tc_comb_uv(ya, yb, prev_out, Wnb, bnb, Wself, bself, Watt, batt):
    wa1 = Watt[:H, :]
    wa2 = Watt[H:, :]
    full = lambda i: (0, 0)
    xa, xb, out, u, v = pl.pallas_call(
        _comb_uv_body,
        grid=(N // RB,),
        in_specs=[
            pl.BlockSpec((NC, RB, HH), lambda i: (0, i, 0)),
            pl.BlockSpec((NC, RB, HH), lambda i: (0, i, 0)),
            pl.BlockSpec((RB, H), lambda i: (i, 0)),
            pl.BlockSpec((H, H), full),
            pl.BlockSpec((1, H), full),
            pl.BlockSpec((H, H), full),
            pl.BlockSpec((1, H), full),
            pl.BlockSpec((H, 1), full),
            pl.BlockSpec((H, 1), full),
            pl.BlockSpec((1, 1), full),
        ],
        out_specs=[
            pl.BlockSpec((RB, HH), lambda i: (i, 0)),
            pl.BlockSpec((RB, HH), lambda i: (i, 0)),
            pl.BlockSpec((RB, H), lambda i: (i, 0)),
            pl.BlockSpec((RB, 1), lambda i: (i, 0)),
            pl.BlockSpec((RB, 1), lambda i: (i, 0)),
        ],
        out_shape=[
            jax.ShapeDtypeStruct((N, HH), jnp.float32),
            jax.ShapeDtypeStruct((N, HH), jnp.float32),
            jax.ShapeDtypeStruct((N, H), jnp.float32),
            jax.ShapeDtypeStruct((N, 1), jnp.float32),
            jax.ShapeDtypeStruct((N, 1), jnp.float32),
        ],
    )(ya, yb, prev_out, Wnb, bnb.reshape(1, H), Wself, bself.reshape(1, H),
      wa1, wa2, batt.reshape(1, 1))
    return xa, xb, out, u.reshape(N), v.reshape(N)


@functools.partial(
    pl.kernel,
    out_type=[
        jax.ShapeDtypeStruct((NW, NCH, CH), jnp.float32),
        jax.ShapeDtypeStruct((NC, NPAD), jnp.float32),
    ],
    mesh=_mesh,
    scratch_types=[
        pltpu.VMEM((N,), jnp.float32),
        pltpu.VMEM((N,), jnp.float32),
        pltpu.VMEM((NCH, CH), jnp.int32),
        pltpu.VMEM((NCH, CH), jnp.int32),
        pltpu.VMEM((NCH, CH), jnp.float32),
        pltpu.VMEM((NPAD // NS,), jnp.float32),
        pltpu.VMEM_SHARED((NPAD,), jnp.float32),
        pltpu.SemaphoreType.DMA,
    ],
    compiler_params=_sc_params,
)
def _sc_mask(u_hbm, v_hbm, row_hbm, col_hbm, mask_hbm, rs_hbm,
             u_v, v_v, row_v, col_v, mask_v, z_v, rs_sh, rsem):
    c = lax.axis_index("c")
    s = lax.axis_index("s")
    w = c * NS + s
    zn = NPAD // NS

    pltpu.sync_copy(u_hbm, u_v)
    pltpu.sync_copy(v_hbm, v_v)
    pltpu.sync_copy(row_hbm.at[w], row_v)
    pltpu.sync_copy(col_hbm.at[w], col_v)

    @pl.loop(0, zn, step=16)
    def _(i):
        z_v[pl.ds(i, 16)] = jnp.zeros((16,), jnp.float32)

    pltpu.sync_copy(z_v, rs_sh.at[pl.ds(s * zn, zn)])
    plsc.subcore_barrier()

    @pl.loop(0, NCH)
    def _(k):
        @plsc.parallel_loop(0, CH, step=16, unroll=2)
        def _(j):
            r = row_v[k, pl.ds(j, 16)]
            cl = col_v[k, pl.ds(j, 16)]
            ug = plsc.load_gather(u_v, [r])
            vg = plsc.load_gather(v_v, [cl])
            gate = 1.0 / (1.0 + jnp.exp(-(ug + vg)))
            m = gate * (ZETA - GAMMA) + GAMMA
            m = jnp.minimum(jnp.maximum(m, 0.0), 1.0)
            mask_v[k, pl.ds(j, 16)] = m
        pltpu.async_copy(mask_v.at[k], rs_sh.at[row_v.at[k]], rsem, add=True)

    @pl.loop(0, NCH)
    def _(k):
        pltpu.make_async_copy(
            mask_v.at[0], rs_sh.at[row_v.at[0]], rsem).wait()

    plsc.subcore_barrier()
    pltpu.sync_copy(rs_sh.at[pl.ds(s * zn, zn)], rs_hbm.at[c, pl.ds(s * zn, zn)])
    pltpu.sync_copy(mask_v, mask_hbm.at[w])


HH = H // 2


@functools.partial(
    pl.kernel,
    out_type=[
        jax.ShapeDtypeStruct((NC, NPAD, HH), jnp.float32),
        jax.ShapeDtypeStruct((NC, NPAD, HH), jnp.float32),
    ],
    mesh=_mesh,
    scratch_types=[
        pltpu.VMEM((NCH, CH), jnp.int32),
        pltpu.VMEM((NCH, CH), jnp.int32),
        pltpu.VMEM((NCH, CH), jnp.float32),
        pltpu.VMEM((NPAD,), jnp.float32),
        pltpu.VMEM((NPAD // NS,), jnp.float32),
        pltpu.VMEM((NPAD // NS,), jnp.float32),
        pltpu.VMEM_SHARED((NPAD,), jnp.float32),
        pltpu.VMEM((CH, HH), jnp.float32),
        pltpu.VMEM((CH, HH), jnp.float32),
        pltpu.VMEM((128, HH), jnp.float32),
        pltpu.VMEM_SHARED((NPAD, HH), jnp.float32),
        pltpu.SemaphoreType.DMA,
        pltpu.SemaphoreType.DMA,
        pltpu.SemaphoreType.DMA,
        pltpu.SemaphoreType.DMA,
    ],
    compiler_params=_sc_params,
)
def _sc_spmm(xa_hbm, xb_hbm, rs_hbm, coef_hbm, row_hbm, col_hbm,
             ya_hbm, yb_hbm, row_v, col_v, coef_v, d_v, rs0_v, rs1_v, d_sh,
             rows0_v, rows1_v, z_v, acc, gs0, gs1, ss0, ss1):
    c = lax.axis_index("c")
    s = lax.axis_index("s")
    w = c * NS + s
    rpw = NPAD // NS

    pltpu.sync_copy(row_hbm.at[w], row_v)
    pltpu.sync_copy(col_hbm.at[w], col_v)
    pltpu.sync_copy(coef_hbm.at[w], coef_v)

    pltpu.sync_copy(rs_hbm.at[0, pl.ds(s * rpw, rpw)], rs0_v)
    pltpu.sync_copy(rs_hbm.at[1, pl.ds(s * rpw, rpw)], rs1_v)

    @plsc.parallel_loop(0, rpw, step=16, unroll=2)
    def _(i):
        sl = pl.ds(i, 16)
        rs = rs0_v[sl] + rs1_v[sl] + 1e-6
        yi = plsc.bitcast(
            jnp.int32(0x5F3759DF) - (plsc.bitcast(rs, jnp.int32) >> 1),
            jnp.float32)
        for _ in range(3):
            yi = yi * (1.5 - 0.5 * rs * yi * yi)
        rs0_v[sl] = jnp.minimum(yi, 10.0)

    pltpu.sync_copy(rs0_v, d_sh.at[pl.ds(s * rpw, rpw)])
    plsc.subcore_barrier()
    pltpu.sync_copy(d_sh, d_v)

    @pl.loop(0, 128)
    def _(i):
        @pl.loop(0, HH, step=16)
        def _(j):
            z_v[i, pl.ds(j, 16)] = jnp.zeros((16,), jnp.float32)

    @pl.loop(0, NCH)
    def _(k):
        @plsc.parallel_loop(0, CH, step=16, unroll=2)
        def _(j):
            r = row_v[k, pl.ds(j, 16)]
            cl = col_v[k, pl.ds(j, 16)]
            dg = plsc.load_gather(d_v, [r]) * plsc.load_gather(d_v, [cl])
            coef_v[k, pl.ds(j, 16)] = coef_v[k, pl.ds(j, 16)] * dg

    def _scale(rows_ref, kk):
        fk = jnp.full((16,), kk, jnp.int32)

        @plsc.parallel_loop(0, CH, step=4, unroll=2)
        def _(e):
            for de in range(4):
                sv = plsc.load_gather(
                    coef_v, [fk, jnp.full((16,), e + de, jnp.int32)])
                for jj in range(HH // 16):
                    sl = pl.ds(jj * 16, 16)
                    rows_ref[e + de, sl] = rows_ref[e + de, sl] * sv

    bufs = ((rows0_v, gs0, ss0), (rows1_v, gs1, ss1))

    def _half(x_hbm, y_hbm):
        for t in range(5):
            pltpu.sync_copy(z_v, acc.at[pl.ds(s * rpw + t * 128, 128), :])
        plsc.subcore_barrier()

        for b, (rows_b, gs_b, ss_b) in enumerate(bufs):
            pltpu.async_copy(x_hbm.at[col_v.at[b]], rows_b, gs_b)

        @pl.loop(0, NCH - 1, step=2)
        def _(k):
            for b, (rows_b, gs_b, ss_b) in enumerate(bufs):
                kk = k + b
                pltpu.make_async_copy(
                    x_hbm.at[col_v.at[kk]], rows_b, gs_b).wait()
                _scale(rows_b, kk)
                pltpu.async_copy(rows_b, acc.at[row_v.at[kk]], ss_b, add=True)

                @pl.when(kk + 2 < NCH)
                def _():
                    pltpu.make_async_copy(
                        rows_b, acc.at[row_v.at[kk]], ss_b).wait()
                    pltpu.async_copy(x_hbm.at[col_v.at[kk + 2]], rows_b, gs_b)

        kk = NCH - 1
        pltpu.make_async_copy(x_hbm.at[col_v.at[kk]], rows0_v, gs0).wait()
        _scale(rows0_v, kk)
        pltpu.async_copy(rows0_v, acc.at[row_v.at[kk]], ss0, add=True)
        pltpu.make_async_copy(rows0_v, acc.at[row_v.at[0]], ss0).wait()
        pltpu.make_async_copy(rows1_v, acc.at[row_v.at[0]], ss1).wait()

        plsc.subcore_barrier()
        for t in range(5):
            sl = pl.ds(s * rpw + t * 128, 128)
            pltpu.sync_copy(acc.at[sl, :], y_hbm.at[c, sl, :])

    _half(xa_hbm, ya_hbm)
    plsc.subcore_barrier()
    _half(xb_hbm, yb_hbm)


def kernel(features, edge_index, Wnb0, bnb0, Wself0, bself0, Watt0, batt0,
           Wnb1, bnb1, Wself1, bself1, Watt1, batt1):
    row3 = edge_index[0].reshape(NW, NCH, CH)
    col3 = edge_index[1].reshape(NW, NCH, CH)
    u, v = _tc_uv(features, Wnb0, bnb0, Wself0, bself0, Watt0, batt0)
    mask3, rs = _sc_mask(u, v, row3, col3)
    ya, yb = _sc_spmm(features[:, :HH], features[:, HH:], rs, mask3,
                      row3, col3)
    xa, xb, out, u2, v2 = _tc_comb_uv(ya, yb, features, Wnb1, bnb1,
                                      Wself1, bself1, Watt1, batt1)
    mask3, rs = _sc_mask(u2, v2, row3, col3)
    ya, yb = _sc_spmm(xa, xb, rs, mask3, row3, col3)
    _, out = _tc_combine(ya, yb, out)
    return out


# speedup vs baseline: 1.3230x; 1.1628x over previous
import dataclasses
import functools

import jax
import jax.numpy as jnp
from jax import lax
from jax.experimental import pallas as pl
from jax.experimental.pallas import tpu as pltpu
from jax.experimental.pallas import tpu_sc as plsc

H = 128
N = 10000
E = 320000
NPAD = 10240
GAMMA = -0.45
ZETA = 1.05
NC = 2
NS = 16
NW = NC * NS
EPW = E // NW
CH = 80
NCH = EPW // CH
RB = 1000

_mesh = plsc.VectorSubcoreMesh(
    core_axis_name="c", subcore_axis_name="s", num_cores=NC, num_subcores=NS)

_sc_params = pltpu.CompilerParams(
    needs_layout_passes=False, use_tc_tiling_on_sc=False)


def _uv_body(x_ref, wnb_ref, bnb_ref, wself_ref, bself_ref, wa1_ref,
             wa2_ref, batt_ref, u_ref, v_ref):
    x = x_ref[...]
    a = jnp.maximum(x @ wnb_ref[...] + bnb_ref[...], 0.0)
    b = jnp.maximum(x @ wself_ref[...] + bself_ref[...], 0.0)
    u_ref[...] = a @ wa1_ref[...]
    v_ref[...] = b @ wa2_ref[...] + batt_ref[...]


def _tc_uv(x, Wnb, bnb, Wself, bself, Watt, batt):
    wa1 = Watt[:H, :]
    wa2 = Watt[H:, :]
    u, v = pl.pallas_call(
        _uv_body,
        grid=(N // RB,),
        in_specs=[
            pl.BlockSpec((RB, H), lambda i: (i, 0)),
            pl.BlockSpec((H, H), lambda i: (0, 0)),
            pl.BlockSpec((1, H), lambda i: (0, 0)),
            pl.BlockSpec((H, H), lambda i: (0, 0)),
            pl.BlockSpec((1, H), lambda i: (0, 0)),
            pl.BlockSpec((H, 1), lambda i: (0, 0)),
            pl.BlockSpec((H, 1), lambda i: (0, 0)),
            pl.BlockSpec((1, 1), lambda i: (0, 0)),
        ],
        out_specs=[
            pl.BlockSpec((RB, 1), lambda i: (i, 0)),
            pl.BlockSpec((RB, 1), lambda i: (i, 0)),
        ],
        out_shape=[
            jax.ShapeDtypeStruct((N, 1), jnp.float32),
            jax.ShapeDtypeStruct((N, 1), jnp.float32),
        ],
    )(x, Wnb, bnb.reshape(1, H), Wself, bself.reshape(1, H), wa1, wa2,
      batt.reshape(1, 1))
    return u.reshape(N), v.reshape(N)


def _comb_body(ya_ref, yb_ref, prev_ref, x_ref, out_ref):
    xn = jnp.concatenate(
        [ya_ref[0] + ya_ref[1], yb_ref[0] + yb_ref[1]], axis=-1)
    x_ref[...] = xn
    out_ref[...] = prev_ref[...] + xn


def _tc_combine(ya, yb, prev_out):
    return pl.pallas_call(
        _comb_body,
        grid=(N // RB,),
        in_specs=[
            pl.BlockSpec((NC, RB, HH), lambda i: (0, i, 0)),
            pl.BlockSpec((NC, RB, HH), lambda i: (0, i, 0)),
            pl.BlockSpec((RB, H), lambda i: (i, 0)),
        ],
        out_specs=[
            pl.BlockSpec((RB, H), lambda i: (i, 0)),
            pl.BlockSpec((RB, H), lambda i: (i, 0)),
        ],
        out_shape=[
            jax.ShapeDtypeStruct((N, H), jnp.float32),
            jax.ShapeDtypeStruct((N, H), jnp.float32),
        ],
    )(ya, yb, prev_out)


def _comb_uv_body(ya_ref, yb_ref, prev_ref, wnb_ref, bnb_ref, wself_ref,
                  bself_ref, wa1_ref, wa2_ref, batt_ref,
                  xa_ref, xb_ref, out_ref, u_ref, v_ref):
    xa = ya_ref[0] + ya_ref[1]
    xb = yb_ref[0] + yb_ref[1]
    xa_ref[...] = xa
    xb_ref[...] = xb
    xn = jnp.concatenate([xa, xb], axis=-1)
    out_ref[...] = prev_ref[...] + xn
    a = jnp.maximum(xn @ wnb_ref[...] + bnb_ref[...], 0.0)
    b = jnp.maximum(xn @ wself_ref[...] + bself_ref[...], 0.0)
    u_ref[...] = a @ wa1_ref[...]
    v_ref[...] = b @ wa2_ref[...] + batt_ref[...]


def _tc_comb_uv(ya, yb, prev_out, Wnb, bnb, Wself, bself, Watt, batt):
    wa1 = Watt[:H, :]
    wa2 = Watt[H:, :]
    full = lambda i: (0, 0)
    xa, xb, out, u, v = pl.pallas_call(
        _comb_uv_body,
        grid=(N // RB,),
        in_specs=[
            pl.BlockSpec((NC, RB, HH), lambda i: (0, i, 0)),
            pl.BlockSpec((NC, RB, HH), lambda i: (0, i, 0)),
            pl.BlockSpec((RB, H), lambda i: (i, 0)),
            pl.BlockSpec((H, H), full),
            pl.BlockSpec((1, H), full),
            pl.BlockSpec((H, H), full),
            pl.BlockSpec((1, H), full),
            pl.BlockSpec((H, 1), full),
            pl.BlockSpec((H, 1), full),
            pl.BlockSpec((1, 1), full),
        ],
        out_specs=[
            pl.BlockSpec((RB, HH), lambda i: (i, 0)),
            pl.BlockSpec((RB, HH), lambda i: (i, 0)),
            pl.BlockSpec((RB, H), lambda i: (i, 0)),
            pl.BlockSpec((RB, 1), lambda i: (i, 0)),
            pl.BlockSpec((RB, 1), lambda i: (i, 0)),
        ],
        out_shape=[
            jax.ShapeDtypeStruct((N, HH), jnp.float32),
            jax.ShapeDtypeStruct((N, HH), jnp.float32),
            jax.ShapeDtypeStruct((N, H), jnp.float32),
            jax.ShapeDtypeStruct((N, 1), jnp.float32),
            jax.ShapeDtypeStruct((N, 1), jnp.float32),
        ],
    )(ya, yb, prev_out, Wnb, bnb.reshape(1, H), Wself, bself.reshape(1, H),
      wa1, wa2, batt.reshape(1, 1))
    return xa, xb, out, u.reshape(N), v.reshape(N)


@functools.partial(
    pl.kernel,
    out_type=[
        jax.ShapeDtypeStruct((NW, NCH, CH), jnp.float32),
        jax.ShapeDtypeStruct((NC, NPAD), jnp.float32),
    ],
    mesh=_mesh,
    scratch_types=[
        pltpu.VMEM((N,), jnp.float32),
        pltpu.VMEM((N,), jnp.float32),
        pltpu.VMEM((NCH, CH), jnp.int32),
        pltpu.VMEM((NCH, CH), jnp.int32),
        pltpu.VMEM((NCH, CH), jnp.float32),
        pltpu.VMEM((NPAD // NS,), jnp.float32),
        pltpu.VMEM_SHARED((NPAD,), jnp.float32),
        pltpu.SemaphoreType.DMA,
    ],
    compiler_params=_sc_params,
)
def _sc_mask(u_hbm, v_hbm, row_hbm, col_hbm, mask_hbm, rs_hbm,
             u_v, v_v, row_v, col_v, mask_v, z_v, rs_sh, rsem):
    c = lax.axis_index("c")
    s = lax.axis_index("s")
    w = c * NS + s
    zn = NPAD // NS

    pltpu.sync_copy(u_hbm, u_v)
    pltpu.sync_copy(v_hbm, v_v)
    pltpu.sync_copy(row_hbm.at[w], row_v)
    pltpu.sync_copy(col_hbm.at[w], col_v)

    @pl.loop(0, zn, step=16)
    def _(i):
        z_v[pl.ds(i, 16)] = jnp.zeros((16,), jnp.float32)

    pltpu.sync_copy(z_v, rs_sh.at[pl.ds(s * zn, zn)])
    plsc.subcore_barrier()

    @pl.loop(0, NCH)
    def _(k):
        @plsc.parallel_loop(0, CH, step=16, unroll=2)
        def _(j):
            r = row_v[k, pl.ds(j, 16)]
            cl = col_v[k, pl.ds(j, 16)]
            ug = plsc.load_gather(u_v, [r])
            vg = plsc.load_gather(v_v, [cl])
            gate = 1.0 / (1.0 + jnp.exp(-(ug + vg)))
            m = gate * (ZETA - GAMMA) + GAMMA
            m = jnp.minimum(jnp.maximum(m, 0.0), 1.0)
            mask_v[k, pl.ds(j, 16)] = m
        pltpu.async_copy(mask_v.at[k], rs_sh.at[row_v.at[k]], rsem, add=True)

    @pl.loop(0, NCH)
    def _(k):
        pltpu.make_async_copy(
            mask_v.at[0], rs_sh.at[row_v.at[0]], rsem).wait()

    plsc.subcore_barrier()
    pltpu.sync_copy(rs_sh.at[pl.ds(s * zn, zn)], rs_hbm.at[c, pl.ds(s * zn, zn)])
    pltpu.sync_copy(mask_v, mask_hbm.at[w])


HH = H // 2


@functools.partial(
    pl.kernel,
    out_type=[
        jax.ShapeDtypeStruct((NC, NPAD, HH), jnp.float32),
        jax.ShapeDtypeStruct((NC, NPAD, HH), jnp.float32),
    ],
    mesh=_mesh,
    scratch_types=[
        pltpu.VMEM((NCH, CH), jnp.int32),
        pltpu.VMEM((NCH, CH), jnp.int32),
        pltpu.VMEM((NCH, CH), jnp.float32),
        pltpu.VMEM((NPAD,), jnp.float32),
        pltpu.VMEM((NPAD // NS,), jnp.float32),
        pltpu.VMEM((NPAD // NS,), jnp.float32),
        pltpu.VMEM_SHARED((NPAD,), jnp.float32),
        pltpu.VMEM((CH, HH), jnp.float32),
        pltpu.VMEM((CH, HH), jnp.float32),
        pltpu.VMEM((CH, HH), jnp.float32),
        pltpu.VMEM((CH, HH), jnp.float32),
        pltpu.VMEM((128, HH), jnp.float32),
        pltpu.VMEM_SHARED((NPAD, HH), jnp.float32),
        pltpu.SemaphoreType.DMA,
        pltpu.SemaphoreType.DMA,
        pltpu.SemaphoreType.DMA,
        pltpu.SemaphoreType.DMA,
        pltpu.SemaphoreType.DMA,
        pltpu.SemaphoreType.DMA,
        pltpu.SemaphoreType.DMA,
        pltpu.SemaphoreType.DMA,
    ],
    compiler_params=_sc_params,
)
def _sc_spmm(xa_hbm, xb_hbm, rs_hbm, coef_hbm, row_hbm, col_hbm,
             ya_hbm, yb_hbm, row_v, col_v, coef_v, d_v, rs0_v, rs1_v, d_sh,
             rows0_v, rows1_v, rows2_v, rows3_v, z_v, acc,
             gs0, gs1, gs2, gs3, ss0, ss1, ss2, ss3):
    c = lax.axis_index("c")
    s = lax.axis_index("s")
    w = c * NS + s
    rpw = NPAD // NS

    pltpu.sync_copy(row_hbm.at[w], row_v)
    pltpu.sync_copy(col_hbm.at[w], col_v)
    pltpu.sync_copy(coef_hbm.at[w], coef_v)

    pltpu.sync_copy(rs_hbm.at[0, pl.ds(s * rpw, rpw)], rs0_v)
    pltpu.sync_copy(rs_hbm.at[1, pl.ds(s * rpw, rpw)], rs1_v)

    @plsc.parallel_loop(0, rpw, step=16, unroll=2)
    def _(i):
        sl = pl.ds(i, 16)
        rs = rs0_v[sl] + rs1_v[sl] + 1e-6
        yi = plsc.bitcast(
            jnp.int32(0x5F3759DF) - (plsc.bitcast(rs, jnp.int32) >> 1),
            jnp.float32)
        for _ in range(3):
            yi = yi * (1.5 - 0.5 * rs * yi * yi)
        rs0_v[sl] = jnp.minimum(yi, 10.0)

    pltpu.sync_copy(rs0_v, d_sh.at[pl.ds(s * rpw, rpw)])
    plsc.subcore_barrier()
    pltpu.sync_copy(d_sh, d_v)

    @pl.loop(0, 128)
    def _(i):
        @pl.loop(0, HH, step=16)
        def _(j):
            z_v[i, pl.ds(j, 16)] = jnp.zeros((16,), jnp.float32)

    @pl.loop(0, NCH)
    def _(k):
        @plsc.parallel_loop(0, CH, step=16, unroll=2)
        def _(j):
            r = row_v[k, pl.ds(j, 16)]
            cl = col_v[k, pl.ds(j, 16)]
            dg = plsc.load_gather(d_v, [r]) * plsc.load_gather(d_v, [cl])
            coef_v[k, pl.ds(j, 16)] = coef_v[k, pl.ds(j, 16)] * dg

    def _scale(rows_ref, kk):
        fk = jnp.full((16,), kk, jnp.int32)

        @plsc.parallel_loop(0, CH, step=4, unroll=2)
        def _(e):
            for de in range(4):
                sv = plsc.load_gather(
                    coef_v, [fk, jnp.full((16,), e + de, jnp.int32)])
                for jj in range(HH // 16):
                    sl = pl.ds(jj * 16, 16)
                    rows_ref[e + de, sl] = rows_ref[e + de, sl] * sv

    bufs = ((rows0_v, gs0, ss0), (rows1_v, gs1, ss1),
            (rows2_v, gs2, ss2), (rows3_v, gs3, ss3))
    NB = len(bufs)

    def _half(x_hbm, y_hbm):
        for t in range(5):
            pltpu.sync_copy(z_v, acc.at[pl.ds(s * rpw + t * 128, 128), :])
        plsc.subcore_barrier()

        for b, (rows_b, gs_b, ss_b) in enumerate(bufs):
            pltpu.async_copy(x_hbm.at[col_v.at[b]], rows_b, gs_b)

        @pl.loop(0, NCH - 1, step=NB)
        def _(k):
            for b, (rows_b, gs_b, ss_b) in enumerate(bufs):
                kk = k + b
                pltpu.make_async_copy(
                    x_hbm.at[col_v.at[kk]], rows_b, gs_b).wait()
                _scale(rows_b, kk)
                pltpu.async_copy(rows_b, acc.at[row_v.at[kk]], ss_b, add=True)

                @pl.when(kk + NB < NCH)
                def _():
                    pltpu.make_async_copy(
                        rows_b, acc.at[row_v.at[kk]], ss_b).wait()
                    pltpu.async_copy(x_hbm.at[col_v.at[kk + NB]], rows_b, gs_b)

        kk = NCH - 1
        pltpu.make_async_copy(x_hbm.at[col_v.at[kk]], rows0_v, gs0).wait()
        _scale(rows0_v, kk)
        pltpu.async_copy(rows0_v, acc.at[row_v.at[kk]], ss0, add=True)
        for b, (rows_b, gs_b, ss_b) in enumerate(bufs):
            pltpu.make_async_copy(rows_b, acc.at[row_v.at[0]], ss_b).wait()

        plsc.subcore_barrier()
        for t in range(5):
            sl = pl.ds(s * rpw + t * 128, 128)
            pltpu.sync_copy(acc.at[sl, :], y_hbm.at[c, sl, :])

    _half(xa_hbm, ya_hbm)
    plsc.subcore_barrier()
    _half(xb_hbm, yb_hbm)


def kernel(features, edge_index, Wnb0, bnb0, Wself0, bself0, Watt0, batt0,
           Wnb1, bnb1, Wself1, bself1, Watt1, batt1):
    row3 = edge_index[0].reshape(NW, NCH, CH)
    col3 = edge_index[1].reshape(NW, NCH, CH)
    u, v = _tc_uv(features, Wnb0, bnb0, Wself0, bself0, Watt0, batt0)
    mask3, rs = _sc_mask(u, v, row3, col3)
    ya, yb = _sc_spmm(features[:, :HH], features[:, HH:], rs, mask3,
                      row3, col3)
    xa, xb, out, u2, v2 = _tc_comb_uv(ya, yb, features, Wnb1, bnb1,
                                      Wself1, bself1, Watt1, batt1)
    mask3, rs = _sc_mask(u2, v2, row3, col3)
    ya, yb = _sc_spmm(xa, xb, rs, mask3, row3, col3)
    _, out = _tc_combine(ya, yb, out)
    return out


# speedup vs baseline: 1.5791x; 1.1936x over previous
import dataclasses
import functools

import numpy as np

import jax
import jax.numpy as jnp
from jax import lax
from jax.experimental import pallas as pl
from jax.experimental.pallas import tpu as pltpu
from jax.experimental.pallas import tpu_sc as plsc

H = 128
N = 10000
E = 320000
NPAD = 10240
GAMMA = -0.45
ZETA = 1.05
NC = 2
NS = 16
NW = NC * NS
EPW = E // NW
CH = 80
NCH = EPW // CH
RB = 1000

_mesh = plsc.VectorSubcoreMesh(
    core_axis_name="c", subcore_axis_name="s", num_cores=NC, num_subcores=NS)

_sc_params = pltpu.CompilerParams(
    needs_layout_passes=False, use_tc_tiling_on_sc=False)


def _uv_body(x_ref, wnb_ref, bnb_ref, wself_ref, bself_ref, wa1_ref,
             wa2_ref, batt_ref, u_ref, v_ref):
    x = x_ref[...]
    a = jnp.maximum(x @ wnb_ref[...] + bnb_ref[...], 0.0)
    b = jnp.maximum(x @ wself_ref[...] + bself_ref[...], 0.0)
    u_ref[...] = a @ wa1_ref[...]
    v_ref[...] = b @ wa2_ref[...] + batt_ref[...]


def _tc_uv(x, Wnb, bnb, Wself, bself, Watt, batt):
    wa1 = Watt[:H, :]
    wa2 = Watt[H:, :]
    u, v = pl.pallas_call(
        _uv_body,
        grid=(N // RB,),
        in_specs=[
            pl.BlockSpec((RB, H), lambda i: (i, 0)),
            pl.BlockSpec((H, H), lambda i: (0, 0)),
            pl.BlockSpec((1, H), lambda i: (0, 0)),
            pl.BlockSpec((H, H), lambda i: (0, 0)),
            pl.BlockSpec((1, H), lambda i: (0, 0)),
            pl.BlockSpec((H, 1), lambda i: (0, 0)),
            pl.BlockSpec((H, 1), lambda i: (0, 0)),
            pl.BlockSpec((1, 1), lambda i: (0, 0)),
        ],
        out_specs=[
            pl.BlockSpec((RB, 1), lambda i: (i, 0)),
            pl.BlockSpec((RB, 1), lambda i: (i, 0)),
        ],
        out_shape=[
            jax.ShapeDtypeStruct((N, 1), jnp.float32),
            jax.ShapeDtypeStruct((N, 1), jnp.float32),
        ],
    )(x, Wnb, bnb.reshape(1, H), Wself, bself.reshape(1, H), wa1, wa2,
      batt.reshape(1, 1))
    return u.reshape(N), v.reshape(N)


def _comb_body(ya_ref, yb_ref, prev_ref, x_ref, out_ref):
    xn = jnp.concatenate(
        [ya_ref[0] + ya_ref[1], yb_ref[0] + yb_ref[1]], axis=-1)
    x_ref[...] = xn
    out_ref[...] = prev_ref[...] + xn


def _tc_combine(ya, yb, prev_out):
    return pl.pallas_call(
        _comb_body,
        grid=(N // RB,),
        in_specs=[
            pl.BlockSpec((NC, RB, HH), lambda i: (0, i, 0)),
            pl.BlockSpec((NC, RB, HH), lambda i: (0, i, 0)),
            pl.BlockSpec((RB, H), lambda i: (i, 0)),
        ],
        out_specs=[
            pl.BlockSpec((RB, H), lambda i: (i, 0)),
            pl.BlockSpec((RB, H), lambda i: (i, 0)),
        ],
        out_shape=[
            jax.ShapeDtypeStruct((N, H), jnp.float32),
            jax.ShapeDtypeStruct((N, H), jnp.float32),
        ],
    )(ya, yb, prev_out)


def _comb_uv_body(ya_ref, yb_ref, prev_ref, wnb_ref, bnb_ref, wself_ref,
                  bself_ref, wa1_ref, wa2_ref, batt_ref,
                  xa_ref, xb_ref, out_ref, u_ref, v_ref):
    xa = ya_ref[0] + ya_ref[1]
    xb = yb_ref[0] + yb_ref[1]
    xa_ref[...] = xa
    xb_ref[...] = xb
    xn = jnp.concatenate([xa, xb], axis=-1)
    out_ref[...] = prev_ref[...] + xn
    a = jnp.maximum(xn @ wnb_ref[...] + bnb_ref[...], 0.0)
    b = jnp.maximum(xn @ wself_ref[...] + bself_ref[...], 0.0)
    u_ref[...] = a @ wa1_ref[...]
    v_ref[...] = b @ wa2_ref[...] + batt_ref[...]


def _tc_comb_uv(ya, yb, prev_out, Wnb, bnb, Wself, bself, Watt, batt):
    wa1 = Watt[:H, :]
    wa2 = Watt[H:, :]
    full = lambda i: (0, 0)
    xa, xb, out, u, v = pl.pallas_call(
        _comb_uv_body,
        grid=(N // RB,),
        in_specs=[
            pl.BlockSpec((NC, RB, HH), lambda i: (0, i, 0)),
            pl.BlockSpec((NC, RB, HH), lambda i: (0, i, 0)),
            pl.BlockSpec((RB, H), lambda i: (i, 0)),
            pl.BlockSpec((H, H), full),
            pl.BlockSpec((1, H), full),
            pl.BlockSpec((H, H), full),
            pl.BlockSpec((1, H), full),
            pl.BlockSpec((H, 1), full),
            pl.BlockSpec((H, 1), full),
            pl.BlockSpec((1, 1), full),
        ],
        out_specs=[
            pl.BlockSpec((RB, HH), lambda i: (i, 0)),
            pl.BlockSpec((RB, HH), lambda i: (i, 0)),
            pl.BlockSpec((RB, H), lambda i: (i, 0)),
            pl.BlockSpec((RB, 1), lambda i: (i, 0)),
            pl.BlockSpec((RB, 1), lambda i: (i, 0)),
        ],
        out_shape=[
            jax.ShapeDtypeStruct((N, HH), jnp.float32),
            jax.ShapeDtypeStruct((N, HH), jnp.float32),
            jax.ShapeDtypeStruct((N, H), jnp.float32),
            jax.ShapeDtypeStruct((N, 1), jnp.float32),
            jax.ShapeDtypeStruct((N, 1), jnp.float32),
        ],
    )(ya, yb, prev_out, Wnb, bnb.reshape(1, H), Wself, bself.reshape(1, H),
      wa1, wa2, batt.reshape(1, 1))
    return xa, xb, out, u.reshape(N), v.reshape(N)


@functools.partial(
    pl.kernel,
    out_type=[
        jax.ShapeDtypeStruct((NW, NCH, CH), jnp.float32),
        jax.ShapeDtypeStruct((NC, NPAD), jnp.float32),
    ],
    mesh=_mesh,
    scratch_types=[
        pltpu.VMEM((N,), jnp.float32),
        pltpu.VMEM((N,), jnp.float32),
        pltpu.VMEM((NCH, CH), jnp.int32),
        pltpu.VMEM((NCH, CH), jnp.int32),
        pltpu.VMEM((NCH, CH), jnp.float32),
        pltpu.VMEM((NPAD // NS,), jnp.float32),
        pltpu.VMEM_SHARED((NPAD,), jnp.float32),
        pltpu.SemaphoreType.DMA,
    ],
    compiler_params=_sc_params,
)
def _sc_mask(u_hbm, v_hbm, row_hbm, col_hbm, mask_hbm, rs_hbm,
             u_v, v_v, row_v, col_v, mask_v, z_v, rs_sh, rsem):
    c = lax.axis_index("c")
    s = lax.axis_index("s")
    w = c * NS + s
    zn = NPAD // NS

    pltpu.sync_copy(u_hbm, u_v)
    pltpu.sync_copy(v_hbm, v_v)
    pltpu.sync_copy(row_hbm.at[w], row_v)
    pltpu.sync_copy(col_hbm.at[w], col_v)

    @pl.loop(0, zn, step=16)
    def _(i):
        z_v[pl.ds(i, 16)] = jnp.zeros((16,), jnp.float32)

    pltpu.sync_copy(z_v, rs_sh.at[pl.ds(s * zn, zn)])
    plsc.subcore_barrier()

    @pl.loop(0, NCH)
    def _(k):
        @plsc.parallel_loop(0, CH, step=16, unroll=2)
        def _(j):
            r = row_v[k, pl.ds(j, 16)]
            cl = col_v[k, pl.ds(j, 16)]
            ug = plsc.load_gather(u_v, [r])
            vg = plsc.load_gather(v_v, [cl])
            gate = 1.0 / (1.0 + jnp.exp(-(ug + vg)))
            m = gate * (ZETA - GAMMA) + GAMMA
            m = jnp.minimum(jnp.maximum(m, 0.0), 1.0)
            mask_v[k, pl.ds(j, 16)] = m
        pltpu.async_copy(mask_v.at[k], rs_sh.at[row_v.at[k]], rsem, add=True)

    @pl.loop(0, NCH)
    def _(k):
        pltpu.make_async_copy(
            mask_v.at[0], rs_sh.at[row_v.at[0]], rsem).wait()

    plsc.subcore_barrier()
    pltpu.sync_copy(rs_sh.at[pl.ds(s * zn, zn)], rs_hbm.at[c, pl.ds(s * zn, zn)])
    pltpu.sync_copy(mask_v, mask_hbm.at[w])


HH = H // 2

_P32 = np.empty(32, np.int32)
_P32[0::2] = np.arange(16)
_P32[1::2] = 16 + np.arange(16)
_PERM = np.concatenate([g * 32 + _P32 for g in range(HH // 32)])


def _half_to_bf16(xh):
    return xh.astype(jnp.bfloat16)[:, _PERM]


@functools.partial(
    pl.kernel,
    out_type=[
        jax.ShapeDtypeStruct((NC, NPAD, HH), jnp.float32),
        jax.ShapeDtypeStruct((NC, NPAD, HH), jnp.float32),
    ],
    mesh=_mesh,
    scratch_types=[
        pltpu.VMEM((NCH, CH), jnp.int32),
        pltpu.VMEM((NCH, CH), jnp.int32),
        pltpu.VMEM((NCH, CH), jnp.float32),
        pltpu.VMEM((NPAD,), jnp.float32),
        pltpu.VMEM((NPAD // NS,), jnp.float32),
        pltpu.VMEM((NPAD // NS,), jnp.float32),
        pltpu.VMEM_SHARED((NPAD,), jnp.float32),
        pltpu.VMEM((CH, HH), jnp.bfloat16),
        pltpu.VMEM((CH, HH), jnp.bfloat16),
        pltpu.VMEM((CH, HH), jnp.bfloat16),
        pltpu.VMEM((CH, HH), jnp.bfloat16),
        pltpu.VMEM((CH, HH), jnp.float32),
        pltpu.VMEM((CH, HH), jnp.float32),
        pltpu.VMEM((CH, HH), jnp.float32),
        pltpu.VMEM((CH, HH), jnp.float32),
        pltpu.VMEM((128, HH), jnp.float32),
        pltpu.VMEM_SHARED((NPAD, HH), jnp.float32),
        pltpu.SemaphoreType.DMA,
        pltpu.SemaphoreType.DMA,
        pltpu.SemaphoreType.DMA,
        pltpu.SemaphoreType.DMA,
        pltpu.SemaphoreType.DMA,
        pltpu.SemaphoreType.DMA,
        pltpu.SemaphoreType.DMA,
        pltpu.SemaphoreType.DMA,
    ],
    compiler_params=_sc_params,
)
def _sc_spmm(xa_hbm, xb_hbm, rs_hbm, coef_hbm, row_hbm, col_hbm,
             ya_hbm, yb_hbm, row_v, col_v, coef_v, d_v, rs0_v, rs1_v, d_sh,
             rows0_v, rows1_v, rows2_v, rows3_v,
             st0_v, st1_v, st2_v, st3_v, z_v, acc,
             gs0, gs1, gs2, gs3, ss0, ss1, ss2, ss3):
    c = lax.axis_index("c")
    s = lax.axis_index("s")
    w = c * NS + s
    rpw = NPAD // NS

    pltpu.sync_copy(row_hbm.at[w], row_v)
    pltpu.sync_copy(col_hbm.at[w], col_v)
    pltpu.sync_copy(coef_hbm.at[w], coef_v)

    pltpu.sync_copy(rs_hbm.at[0, pl.ds(s * rpw, rpw)], rs0_v)
    pltpu.sync_copy(rs_hbm.at[1, pl.ds(s * rpw, rpw)], rs1_v)

    @plsc.parallel_loop(0, rpw, step=16, unroll=2)
    def _(i):
        sl = pl.ds(i, 16)
        rs = rs0_v[sl] + rs1_v[sl] + 1e-6
        yi = plsc.bitcast(
            jnp.int32(0x5F3759DF) - (plsc.bitcast(rs, jnp.int32) >> 1),
            jnp.float32)
        for _ in range(3):
            yi = yi * (1.5 - 0.5 * rs * yi * yi)
        rs0_v[sl] = jnp.minimum(yi, 10.0)

    pltpu.sync_copy(rs0_v, d_sh.at[pl.ds(s * rpw, rpw)])
    plsc.subcore_barrier()
    pltpu.sync_copy(d_sh, d_v)

    @pl.loop(0, 128)
    def _(i):
        @pl.loop(0, HH, step=16)
        def _(j):
            z_v[i, pl.ds(j, 16)] = jnp.zeros((16,), jnp.float32)

    @pl.loop(0, NCH)
    def _(k):
        @plsc.parallel_loop(0, CH, step=16, unroll=2)
        def _(j):
            r = row_v[k, pl.ds(j, 16)]
            cl = col_v[k, pl.ds(j, 16)]
            dg = plsc.load_gather(d_v, [r]) * plsc.load_gather(d_v, [cl])
            coef_v[k, pl.ds(j, 16)] = coef_v[k, pl.ds(j, 16)] * dg

    def _scale(rows_ref, st_ref, kk):
        fk = jnp.full((16,), kk, jnp.int32)

        @plsc.parallel_loop(0, CH, step=4, unroll=2)
        def _(e):
            for de in range(4):
                sv = plsc.load_gather(
                    coef_v, [fk, jnp.full((16,), e + de, jnp.int32)])
                for g in range(HH // 32):
                    packed = rows_ref[e + de, pl.ds(g * 32, 32)]
                    lo, hi = plsc.unpack(
                        packed, format=plsc.PackFormat.INTERLEAVED,
                        preferred_element_type=jnp.float32)
                    st_ref[e + de, pl.ds(g * 32, 16)] = lo * sv
                    st_ref[e + de, pl.ds(g * 32 + 16, 16)] = hi * sv

    bufs = ((rows0_v, st0_v, gs0, ss0), (rows1_v, st1_v, gs1, ss1),
            (rows2_v, st2_v, gs2, ss2), (rows3_v, st3_v, gs3, ss3))
    NB = len(bufs)

    def _half(x_hbm, y_hbm):
        for t in range(5):
            pltpu.sync_copy(z_v, acc.at[pl.ds(s * rpw + t * 128, 128), :])
        plsc.subcore_barrier()

        for b, (rows_b, st_b, gs_b, ss_b) in enumerate(bufs):
            pltpu.async_copy(x_hbm.at[col_v.at[b]], rows_b, gs_b)

        @pl.loop(0, NCH - 1, step=NB)
        def _(k):
            for b, (rows_b, st_b, gs_b, ss_b) in enumerate(bufs):
                kk = k + b
                pltpu.make_async_copy(
                    x_hbm.at[col_v.at[kk]], rows_b, gs_b).wait()

                @pl.when(kk >= NB)
                def _():
                    pltpu.make_async_copy(
                        st_b, acc.at[row_v.at[0]], ss_b).wait()

                _scale(rows_b, st_b, kk)
                pltpu.async_copy(st_b, acc.at[row_v.at[kk]], ss_b, add=True)

                @pl.when(kk + NB < NCH)
                def _():
                    pltpu.async_copy(x_hbm.at[col_v.at[kk + NB]], rows_b, gs_b)

        kk = NCH - 1
        pltpu.make_async_copy(x_hbm.at[col_v.at[kk]], rows0_v, gs0).wait()
        pltpu.make_async_copy(st0_v, acc.at[row_v.at[0]], ss0).wait()
        _scale(rows0_v, st0_v, kk)
        pltpu.async_copy(st0_v, acc.at[row_v.at[kk]], ss0, add=True)
        for b, (rows_b, st_b, gs_b, ss_b) in enumerate(bufs):
            pltpu.make_async_copy(st_b, acc.at[row_v.at[0]], ss_b).wait()

        plsc.subcore_barrier()
        for t in range(5):
            sl = pl.ds(s * rpw + t * 128, 128)
            pltpu.sync_copy(acc.at[sl, :], y_hbm.at[c, sl, :])

    _half(xa_hbm, ya_hbm)
    plsc.subcore_barrier()
    _half(xb_hbm, yb_hbm)


def kernel(features, edge_index, Wnb0, bnb0, Wself0, bself0, Watt0, batt0,
           Wnb1, bnb1, Wself1, bself1, Watt1, batt1):
    row3 = edge_index[0].reshape(NW, NCH, CH)
    col3 = edge_index[1].reshape(NW, NCH, CH)
    u, v = _tc_uv(features, Wnb0, bnb0, Wself0, bself0, Watt0, batt0)
    mask3, rs = _sc_mask(u, v, row3, col3)
    ya, yb = _sc_spmm(_half_to_bf16(features[:, :HH]),
                      _half_to_bf16(features[:, HH:]), rs, mask3, row3, col3)
    xa, xb, out, u2, v2 = _tc_comb_uv(ya, yb, features, Wnb1, bnb1,
                                      Wself1, bself1, Watt1, batt1)
    mask3, rs = _sc_mask(u2, v2, row3, col3)
    ya, yb = _sc_spmm(_half_to_bf16(xa), _half_to_bf16(xb), rs, mask3,
                      row3, col3)
    _, out = _tc_combine(ya, yb, out)
    return out


# speedup vs baseline: 1.5808x; 1.0010x over previous
import functools

import numpy as np

import jax
import jax.numpy as jnp
from jax import lax
from jax.experimental import pallas as pl
from jax.experimental.pallas import tpu as pltpu
from jax.experimental.pallas import tpu_sc as plsc

H = 128
N = 10000
E = 320000
NPAD = 10240
GAMMA = -0.45
ZETA = 1.05
NC = 2
NS = 16
NW = NC * NS
EPW = E // NW
CH = 80
NCH = EPW // CH
RB = 1000

_mesh = plsc.VectorSubcoreMesh(
    core_axis_name="c", subcore_axis_name="s", num_cores=NC, num_subcores=NS)

_sc_params = pltpu.CompilerParams(
    needs_layout_passes=False, use_tc_tiling_on_sc=False)


def _uv_body(x_ref, wnb_ref, bnb_ref, wself_ref, bself_ref, wa1_ref,
             wa2_ref, batt_ref, u_ref, v_ref):
    x = x_ref[...]
    a = jnp.maximum(x @ wnb_ref[...] + bnb_ref[...], 0.0)
    b = jnp.maximum(x @ wself_ref[...] + bself_ref[...], 0.0)
    u_ref[...] = a @ wa1_ref[...]
    v_ref[...] = b @ wa2_ref[...] + batt_ref[...]


def _tc_uv(x, Wnb, bnb, Wself, bself, Watt, batt):
    wa1 = Watt[:H, :]
    wa2 = Watt[H:, :]
    u, v = pl.pallas_call(
        _uv_body,
        grid=(N // RB,),
        in_specs=[
            pl.BlockSpec((RB, H), lambda i: (i, 0)),
            pl.BlockSpec((H, H), lambda i: (0, 0)),
            pl.BlockSpec((1, H), lambda i: (0, 0)),
            pl.BlockSpec((H, H), lambda i: (0, 0)),
            pl.BlockSpec((1, H), lambda i: (0, 0)),
            pl.BlockSpec((H, 1), lambda i: (0, 0)),
            pl.BlockSpec((H, 1), lambda i: (0, 0)),
            pl.BlockSpec((1, 1), lambda i: (0, 0)),
        ],
        out_specs=[
            pl.BlockSpec((RB, 1), lambda i: (i, 0)),
            pl.BlockSpec((RB, 1), lambda i: (i, 0)),
        ],
        out_shape=[
            jax.ShapeDtypeStruct((N, 1), jnp.float32),
            jax.ShapeDtypeStruct((N, 1), jnp.float32),
        ],
    )(x, Wnb, bnb.reshape(1, H), Wself, bself.reshape(1, H), wa1, wa2,
      batt.reshape(1, 1))
    return u.reshape(N), v.reshape(N)


def _comb_body(ya_ref, yb_ref, prev_ref, x_ref, out_ref):
    xn = jnp.concatenate(
        [ya_ref[0] + ya_ref[1], yb_ref[0] + yb_ref[1]], axis=-1)
    x_ref[...] = xn
    out_ref[...] = prev_ref[...] + xn


def _tc_combine(ya, yb, prev_out):
    return pl.pallas_call(
        _comb_body,
        grid=(N // RB,),
        in_specs=[
            pl.BlockSpec((NC, RB, HH), lambda i: (0, i, 0)),
            pl.BlockSpec((NC, RB, HH), lambda i: (0, i, 0)),
            pl.BlockSpec((RB, H), lambda i: (i, 0)),
        ],
        out_specs=[
            pl.BlockSpec((RB, H), lambda i: (i, 0)),
            pl.BlockSpec((RB, H), lambda i: (i, 0)),
        ],
        out_shape=[
            jax.ShapeDtypeStruct((N, H), jnp.float32),
            jax.ShapeDtypeStruct((N, H), jnp.float32),
        ],
    )(ya, yb, prev_out)


def _comb_uv_body(ya_ref, yb_ref, prev_ref, wnb_ref, bnb_ref, wself_ref,
                  bself_ref, wa1_ref, wa2_ref, batt_ref,
                  xa_ref, xb_ref, out_ref, u_ref, v_ref):
    xa = ya_ref[0] + ya_ref[1]
    xb = yb_ref[0] + yb_ref[1]
    xa_ref[...] = xa
    xb_ref[...] = xb
    xn = jnp.concatenate([xa, xb], axis=-1)
    out_ref[...] = prev_ref[...] + xn
    a = jnp.maximum(xn @ wnb_ref[...] + bnb_ref[...], 0.0)
    b = jnp.maximum(xn @ wself_ref[...] + bself_ref[...], 0.0)
    u_ref[...] = a @ wa1_ref[...]
    v_ref[...] = b @ wa2_ref[...] + batt_ref[...]


def _tc_comb_uv(ya, yb, prev_out, Wnb, bnb, Wself, bself, Watt, batt):
    wa1 = Watt[:H, :]
    wa2 = Watt[H:, :]
    full = lambda i: (0, 0)
    xa, xb, out, u, v = pl.pallas_call(
        _comb_uv_body,
        grid=(N // RB,),
        in_specs=[
            pl.BlockSpec((NC, RB, HH), lambda i: (0, i, 0)),
            pl.BlockSpec((NC, RB, HH), lambda i: (0, i, 0)),
            pl.BlockSpec((RB, H), lambda i: (i, 0)),
            pl.BlockSpec((H, H), full),
            pl.BlockSpec((1, H), full),
            pl.BlockSpec((H, H), full),
            pl.BlockSpec((1, H), full),
            pl.BlockSpec((H, 1), full),
            pl.BlockSpec((H, 1), full),
            pl.BlockSpec((1, 1), full),
        ],
        out_specs=[
            pl.BlockSpec((RB, HH), lambda i: (i, 0)),
            pl.BlockSpec((RB, HH), lambda i: (i, 0)),
            pl.BlockSpec((RB, H), lambda i: (i, 0)),
            pl.BlockSpec((RB, 1), lambda i: (i, 0)),
            pl.BlockSpec((RB, 1), lambda i: (i, 0)),
        ],
        out_shape=[
            jax.ShapeDtypeStruct((N, HH), jnp.float32),
            jax.ShapeDtypeStruct((N, HH), jnp.float32),
            jax.ShapeDtypeStruct((N, H), jnp.float32),
            jax.ShapeDtypeStruct((N, 1), jnp.float32),
            jax.ShapeDtypeStruct((N, 1), jnp.float32),
        ],
    )(ya, yb, prev_out, Wnb, bnb.reshape(1, H), Wself, bself.reshape(1, H),
      wa1, wa2, batt.reshape(1, 1))
    return xa, xb, out, u.reshape(N), v.reshape(N)


@functools.partial(
    pl.kernel,
    out_type=[
        jax.ShapeDtypeStruct((NW, NCH, CH), jnp.float32),
        jax.ShapeDtypeStruct((NC, NPAD), jnp.float32),
    ],
    mesh=_mesh,
    scratch_types=[
        pltpu.VMEM((N,), jnp.float32),
        pltpu.VMEM((N,), jnp.float32),
        pltpu.VMEM((NCH, CH), jnp.int32),
        pltpu.VMEM((NCH, CH), jnp.int32),
        pltpu.VMEM((NCH, CH), jnp.float32),
        pltpu.VMEM((NPAD // NS,), jnp.float32),
        pltpu.VMEM_SHARED((NPAD,), jnp.float32),
        pltpu.SemaphoreType.DMA,
    ],
    compiler_params=_sc_params,
)
def _sc_mask(u_hbm, v_hbm, row_hbm, col_hbm, mask_hbm, rs_hbm,
             u_v, v_v, row_v, col_v, mask_v, z_v, rs_sh, rsem):
    c = lax.axis_index("c")
    s = lax.axis_index("s")
    w = c * NS + s
    zn = NPAD // NS

    pltpu.sync_copy(u_hbm, u_v)
    pltpu.sync_copy(v_hbm, v_v)
    pltpu.sync_copy(row_hbm.at[w], row_v)
    pltpu.sync_copy(col_hbm.at[w], col_v)

    @pl.loop(0, zn, step=16)
    def _(i):
        z_v[pl.ds(i, 16)] = jnp.zeros((16,), jnp.float32)

    pltpu.sync_copy(z_v, rs_sh.at[pl.ds(s * zn, zn)])
    plsc.subcore_barrier()

    @pl.loop(0, NCH)
    def _(k):
        @plsc.parallel_loop(0, CH, step=16, unroll=2)
        def _(j):
            r = row_v[k, pl.ds(j, 16)]
            cl = col_v[k, pl.ds(j, 16)]
            ug = plsc.load_gather(u_v, [r])
            vg = plsc.load_gather(v_v, [cl])
            gate = 1.0 / (1.0 + jnp.exp(-(ug + vg)))
            m = gate * (ZETA - GAMMA) + GAMMA
            m = jnp.minimum(jnp.maximum(m, 0.0), 1.0)
            mask_v[k, pl.ds(j, 16)] = m
        pltpu.async_copy(mask_v.at[k], rs_sh.at[row_v.at[k]], rsem, add=True)

    @pl.loop(0, NCH)
    def _(k):
        pltpu.make_async_copy(
            mask_v.at[0], rs_sh.at[row_v.at[0]], rsem).wait()

    plsc.subcore_barrier()
    pltpu.sync_copy(rs_sh.at[pl.ds(s * zn, zn)], rs_hbm.at[c, pl.ds(s * zn, zn)])
    pltpu.sync_copy(mask_v, mask_hbm.at[w])


HH = H // 2

_P32 = np.empty(32, np.int32)
_P32[0::2] = np.arange(16)
_P32[1::2] = 16 + np.arange(16)
_PERM = np.concatenate([g * 32 + _P32 for g in range(HH // 32)])


def _half_to_bf16(xh):
    return xh.astype(jnp.bfloat16)[:, _PERM]


@functools.partial(
    pl.kernel,
    out_type=[
        jax.ShapeDtypeStruct((NC, NPAD, HH), jnp.float32),
        jax.ShapeDtypeStruct((NC, NPAD, HH), jnp.float32),
    ],
    mesh=_mesh,
    scratch_types=[
        pltpu.VMEM((NCH, CH), jnp.int32),
        pltpu.VMEM((NCH, CH), jnp.int32),
        pltpu.VMEM((NCH, CH), jnp.float32),
        pltpu.VMEM((NPAD,), jnp.float32),
        pltpu.VMEM((NPAD // NS,), jnp.float32),
        pltpu.VMEM((NPAD // NS,), jnp.float32),
        pltpu.VMEM_SHARED((NPAD,), jnp.float32),
        pltpu.VMEM((CH, HH), jnp.bfloat16),
        pltpu.VMEM((CH, HH), jnp.bfloat16),
        pltpu.VMEM((CH, HH), jnp.bfloat16),
        pltpu.VMEM((CH, HH), jnp.bfloat16),
        pltpu.VMEM((CH, HH), jnp.float32),
        pltpu.VMEM((CH, HH), jnp.float32),
        pltpu.VMEM((CH, HH), jnp.float32),
        pltpu.VMEM((CH, HH), jnp.float32),
        pltpu.VMEM((128, HH), jnp.float32),
        pltpu.VMEM_SHARED((NPAD, HH), jnp.float32),
        pltpu.SemaphoreType.DMA,
        pltpu.SemaphoreType.DMA,
        pltpu.SemaphoreType.DMA,
        pltpu.SemaphoreType.DMA,
        pltpu.SemaphoreType.DMA,
        pltpu.SemaphoreType.DMA,
        pltpu.SemaphoreType.DMA,
        pltpu.SemaphoreType.DMA,
    ],
    compiler_params=_sc_params,
)
def _sc_spmm(xa_hbm, xb_hbm, rs_hbm, coef_hbm, row_hbm, col_hbm,
             ya_hbm, yb_hbm, row_v, col_v, coef_v, d_v, rs0_v, rs1_v, d_sh,
             rows0_v, rows1_v, rows2_v, rows3_v,
             st0_v, st1_v, st2_v, st3_v, z_v, acc,
             gs0, gs1, gs2, gs3, ss0, ss1, ss2, ss3):
    c = lax.axis_index("c")
    s = lax.axis_index("s")
    w = c * NS + s
    rpw = NPAD // NS

    pltpu.sync_copy(row_hbm.at[w], row_v)
    pltpu.sync_copy(col_hbm.at[w], col_v)
    pltpu.sync_copy(coef_hbm.at[w], coef_v)

    pltpu.sync_copy(rs_hbm.at[0, pl.ds(s * rpw, rpw)], rs0_v)
    pltpu.sync_copy(rs_hbm.at[1, pl.ds(s * rpw, rpw)], rs1_v)

    @plsc.parallel_loop(0, rpw, step=16, unroll=2)
    def _(i):
        sl = pl.ds(i, 16)
        rs = rs0_v[sl] + rs1_v[sl] + 1e-6
        yi = plsc.bitcast(
            jnp.int32(0x5F3759DF) - (plsc.bitcast(rs, jnp.int32) >> 1),
            jnp.float32)
        for _ in range(3):
            yi = yi * (1.5 - 0.5 * rs * yi * yi)
        rs0_v[sl] = jnp.minimum(yi, 10.0)

    pltpu.sync_copy(rs0_v, d_sh.at[pl.ds(s * rpw, rpw)])
    plsc.subcore_barrier()
    pltpu.sync_copy(d_sh, d_v)

    @pl.loop(0, 128)
    def _(i):
        @pl.loop(0, HH, step=16)
        def _(j):
            z_v[i, pl.ds(j, 16)] = jnp.zeros((16,), jnp.float32)

    @pl.loop(0, NCH)
    def _(k):
        @plsc.parallel_loop(0, CH, step=16, unroll=2)
        def _(j):
            r = row_v[k, pl.ds(j, 16)]
            cl = col_v[k, pl.ds(j, 16)]
            dg = plsc.load_gather(d_v, [r]) * plsc.load_gather(d_v, [cl])
            coef_v[k, pl.ds(j, 16)] = coef_v[k, pl.ds(j, 16)] * dg

    def _scale(rows_ref, st_ref, kk):
        fk = jnp.full((16,), kk, jnp.int32)

        @plsc.parallel_loop(0, CH, step=4, unroll=2)
        def _(e):
            for de in range(4):
                sv = plsc.load_gather(
                    coef_v, [fk, jnp.full((16,), e + de, jnp.int32)])
                for g in range(HH // 32):
                    packed = rows_ref[e + de, pl.ds(g * 32, 32)]
                    lo, hi = plsc.unpack(
                        packed, format=plsc.PackFormat.INTERLEAVED,
                        preferred_element_type=jnp.float32)
                    st_ref[e + de, pl.ds(g * 32, 16)] = lo * sv
                    st_ref[e + de, pl.ds(g * 32 + 16, 16)] = hi * sv

    bufs = ((rows0_v, st0_v, gs0, ss0), (rows1_v, st1_v, gs1, ss1),
            (rows2_v, st2_v, gs2, ss2), (rows3_v, st3_v, gs3, ss3))
    NB = len(bufs)

    def _half(x_hbm, y_hbm):
        for t in range(5):
            pltpu.sync_copy(z_v, acc.at[pl.ds(s * rpw + t * 128, 128), :])
        plsc.subcore_barrier()

        for b, (rows_b, st_b, gs_b, ss_b) in enumerate(bufs):
            pltpu.async_copy(x_hbm.at[col_v.at[b]], rows_b, gs_b)

        @pl.loop(0, NCH - 1, step=NB)
        def _(k):
            for b, (rows_b, st_b, gs_b, ss_b) in enumerate(bufs):
                kk = k + b
                pltpu.make_async_copy(
                    x_hbm.at[col_v.at[kk]], rows_b, gs_b).wait()

                @pl.when(kk >= NB)
                def _():
                    pltpu.make_async_copy(
                        st_b, acc.at[row_v.at[0]], ss_b).wait()

                _scale(rows_b, st_b, kk)
                pltpu.async_copy(st_b, acc.at[row_v.at[kk]], ss_b, add=True)

                @pl.when(kk + NB < NCH)
                def _():
                    pltpu.async_copy(x_hbm.at[col_v.at[kk + NB]], rows_b, gs_b)

        kk = NCH - 1
        pltpu.make_async_copy(x_hbm.at[col_v.at[kk]], rows0_v, gs0).wait()
        pltpu.make_async_copy(st0_v, acc.at[row_v.at[0]], ss0).wait()
        _scale(rows0_v, st0_v, kk)
        pltpu.async_copy(st0_v, acc.at[row_v.at[kk]], ss0, add=True)
        for b, (rows_b, st_b, gs_b, ss_b) in enumerate(bufs):
            pltpu.make_async_copy(st_b, acc.at[row_v.at[0]], ss_b).wait()

        plsc.subcore_barrier()
        for t in range(5):
            sl = pl.ds(s * rpw + t * 128, 128)
            pltpu.sync_copy(acc.at[sl, :], y_hbm.at[c, sl, :])

    _half(xa_hbm, ya_hbm)
    plsc.subcore_barrier()
    _half(xb_hbm, yb_hbm)


def kernel(features, edge_index, Wnb0, bnb0, Wself0, bself0, Watt0, batt0,
           Wnb1, bnb1, Wself1, bself1, Watt1, batt1):
    row3 = edge_index[0].reshape(NW, NCH, CH)
    col3 = edge_index[1].reshape(NW, NCH, CH)
    u, v = _tc_uv(features, Wnb0, bnb0, Wself0, bself0, Watt0, batt0)
    mask3, rs = _sc_mask(u, v, row3, col3)
    ya, yb = _sc_spmm(_half_to_bf16(features[:, :HH]),
                      _half_to_bf16(features[:, HH:]), rs, mask3, row3, col3)
    xa, xb, out, u2, v2 = _tc_comb_uv(ya, yb, features, Wnb1, bnb1,
                                      Wself1, bself1, Watt1, batt1)
    mask3, rs = _sc_mask(u2, v2, row3, col3)
    ya, yb = _sc_spmm(_half_to_bf16(xa), _half_to_bf16(xb), rs, mask3,
                      row3, col3)
    _, out = _tc_combine(ya, yb, out)
    return out
